# baseline (speedup 1.0000x reference)
import dataclasses
import functools

import jax
import jax.numpy as jnp
from jax.experimental import pallas as pl
from jax.experimental.pallas import tpu as pltpu
from jax.experimental.pallas import tpu_sc as plsc

_SC_CHUNK = 4096
_LANES = 16
_BATCH_BLOCK = 128


def _sc_bucketize(t_flat):
    n = t_flat.shape[1]
    mesh = plsc.VectorSubcoreMesh(core_axis_name="c", subcore_axis_name="s")
    cp = pltpu.CompilerParams()
    if "needs_layout_passes" in pltpu.CompilerParams.__dataclass_fields__:
        cp = dataclasses.replace(cp, needs_layout_passes=False)

    @functools.partial(
        pl.kernel,
        out_type=jax.ShapeDtypeStruct((1, n), jnp.int32),
        mesh=mesh,
        compiler_params=cp,
    )
    def sc_idx(t_hbm, o_hbm):
        def body(t_vmem, o_vmem):
            tv = t_vmem.at[0]
            ov = o_vmem.at[0]
            for j in range(_SC_CHUNK // _LANES):
                sl = pl.ds(j * _LANES, _LANES)
                v = tv[sl]
                ov[sl] = ((v >= 4.0).astype(jnp.int32)
                          + (v >= 12.0).astype(jnp.int32)
                          + (v >= 24.0).astype(jnp.int32)
                          + (v >= 48.0).astype(jnp.int32))

        pltpu.emit_pipeline(
            body,
            grid=(n // _SC_CHUNK,),
            in_specs=[pl.BlockSpec((1, _SC_CHUNK), lambda i: (0, i))],
            out_specs=[pl.BlockSpec((1, _SC_CHUNK), lambda i: (0, i))],
            core_axis_name=("c", "s"),
            dimension_semantics=(pltpu.PARALLEL,),
        )(t_hbm, o_hbm)

    return sc_idx(t_flat)


def _tc_expand_kernel(idx_ref, w_ref, out_ref):
    b = idx_ref[...][:, :, None]
    w0 = w_ref[0]
    w1 = w_ref[1]
    w2 = w_ref[2]
    w3 = w_ref[3]
    w4 = w_ref[4]
    out_ref[...] = jnp.where(
        b >= 4, w4,
        jnp.where(b >= 3, w3,
                  jnp.where(b >= 2, w2,
                            jnp.where(b >= 1, w1, w0))))


def kernel(visit_rel_times, time_embed_weight):
    batch, hist = visit_rel_times.shape
    _, d = time_embed_weight.shape
    n = batch * hist

    idx = _sc_bucketize(visit_rel_times.reshape(1, n)).reshape(batch, hist)

    rb = _BATCH_BLOCK
    return pl.pallas_call(
        _tc_expand_kernel,
        grid=(batch // rb,),
        in_specs=[
            pl.BlockSpec((rb, hist), lambda i: (i, 0)),
            pl.BlockSpec((5, d), lambda i: (0, 0)),
        ],
        out_specs=pl.BlockSpec((rb, hist, d), lambda i: (i, 0, 0)),
        out_shape=jax.ShapeDtypeStruct((batch, hist, d), jnp.float32),
    )(idx, time_embed_weight)


# speedup vs baseline: 1.0358x; 1.0358x over previous
import dataclasses
import functools

import jax
import jax.numpy as jnp
from jax.experimental import pallas as pl
from jax.experimental.pallas import tpu as pltpu
from jax.experimental.pallas import tpu_sc as plsc

_SC_CHUNK = 2048
_LANES = 16
_BATCH_BLOCK = 128


def _sc_bucketize(t_flat):
    n = t_flat.shape[1]
    mesh = plsc.VectorSubcoreMesh(core_axis_name="c", subcore_axis_name="s")
    cp = pltpu.CompilerParams()
    if "needs_layout_passes" in pltpu.CompilerParams.__dataclass_fields__:
        cp = dataclasses.replace(cp, needs_layout_passes=False)

    @functools.partial(
        pl.kernel,
        out_type=jax.ShapeDtypeStruct((1, n), jnp.int32),
        mesh=mesh,
        compiler_params=cp,
    )
    def sc_idx(t_hbm, o_hbm):
        def body(t_vmem, o_vmem):
            tv = t_vmem.at[0]
            ov = o_vmem.at[0]
            for j in range(_SC_CHUNK // _LANES):
                sl = pl.ds(j * _LANES, _LANES)
                v = tv[sl]
                ov[sl] = ((v >= 4.0).astype(jnp.int32)
                          + (v >= 12.0).astype(jnp.int32)
                          + (v >= 24.0).astype(jnp.int32)
                          + (v >= 48.0).astype(jnp.int32))

        pltpu.emit_pipeline(
            body,
            grid=(n // _SC_CHUNK,),
            in_specs=[pl.BlockSpec((1, _SC_CHUNK), lambda i: (0, i))],
            out_specs=[pl.BlockSpec((1, _SC_CHUNK), lambda i: (0, i))],
            core_axis_name=("c", "s"),
            dimension_semantics=(pltpu.PARALLEL,),
        )(t_hbm, o_hbm)

    return sc_idx(t_flat)


def _tc_expand_kernel(idx_ref, w_ref, out_ref):
    b = idx_ref[...][:, :, None]
    w0 = w_ref[0]
    w1 = w_ref[1]
    w2 = w_ref[2]
    w3 = w_ref[3]
    w4 = w_ref[4]
    out_ref[...] = jnp.where(
        b >= 4, w4,
        jnp.where(b >= 3, w3,
                  jnp.where(b >= 2, w2,
                            jnp.where(b >= 1, w1, w0))))


def kernel(visit_rel_times, time_embed_weight):
    batch, hist = visit_rel_times.shape
    _, d = time_embed_weight.shape
    n = batch * hist

    idx = _sc_bucketize(visit_rel_times.reshape(1, n)).reshape(batch, hist)

    rb = _BATCH_BLOCK
    return pl.pallas_call(
        _tc_expand_kernel,
        grid=(batch // rb,),
        in_specs=[
            pl.BlockSpec((rb, hist), lambda i: (i, 0)),
            pl.BlockSpec((5, d), lambda i: (0, 0)),
        ],
        out_specs=pl.BlockSpec((rb, hist, d), lambda i: (i, 0, 0)),
        out_shape=jax.ShapeDtypeStruct((batch, hist, d), jnp.float32),
    )(idx, time_embed_weight)
